# Initial kernel scaffold; baseline (speedup 1.0000x reference)
import functools

import jax
import jax.numpy as jnp
from jax import lax
from jax.experimental import pallas as pl
from jax.experimental.pallas import tpu as pltpu
from jax.experimental.pallas import tpu_sc as plsc

N = 10000
E = 320000
D = 128

NP = 10240
CHUNK = 128
EP = 327680
NROWS = EP // CHUNK
NSC = 2
NTILE = 16
ROWS_PER_TILE = NROWS // (NSC * NTILE)
NODE_ROWS_PER_TILE = NP // NTILE

_mesh = plsc.VectorSubcoreMesh(core_axis_name="c", subcore_axis_name="s")


@functools.partial(
    pl.kernel,
    mesh=_mesh,
    out_type=jax.ShapeDtypeStruct((NSC, NP, 16), jnp.float32),
    scratch_types=[
        pltpu.VMEM((ROWS_PER_TILE, CHUNK), jnp.int32),
        pltpu.VMEM((CHUNK, 16), jnp.float32),
        pltpu.VMEM_SHARED((NP, 16), jnp.float32),
    ],
)
def _hist_sc(dstr_hbm, zeros16_hbm, ones16_hbm, out_hbm, dst_v, ones_v, deg_sh):
    c = lax.axis_index("c")
    s = lax.axis_index("s")
    node_base = s * NODE_ROWS_PER_TILE
    pltpu.sync_copy(zeros16_hbm.at[pl.ds(node_base, NODE_ROWS_PER_TILE)],
                    deg_sh.at[pl.ds(node_base, NODE_ROWS_PER_TILE)])
    pltpu.sync_copy(ones16_hbm, ones_v)
    edge_base = (c * NTILE + s) * ROWS_PER_TILE
    pltpu.sync_copy(dstr_hbm.at[pl.ds(edge_base, ROWS_PER_TILE)], dst_v)
    plsc.subcore_barrier()

    def body(j, carry):
        pltpu.sync_copy(ones_v, deg_sh.at[dst_v.at[j]], add=True)
        return carry

    lax.fori_loop(0, ROWS_PER_TILE, body, 0)
    plsc.subcore_barrier()
    pltpu.sync_copy(deg_sh.at[pl.ds(node_base, NODE_ROWS_PER_TILE)],
                    out_hbm.at[c, pl.ds(node_base, NODE_ROWS_PER_TILE)])


@functools.partial(
    pl.kernel,
    mesh=_mesh,
    out_type=jax.ShapeDtypeStruct((NSC, NP, D), jnp.float32),
    scratch_types=[
        pltpu.VMEM((ROWS_PER_TILE, CHUNK), jnp.int32),
        pltpu.VMEM((ROWS_PER_TILE, CHUNK), jnp.int32),
        pltpu.VMEM((CHUNK, D), jnp.float32),
        pltpu.VMEM_SHARED((NP, D), jnp.float32),
        pltpu.SemaphoreType.DMA,
    ],
)
def _agg_sc(y_hbm, srcr_hbm, dstr_hbm, zerosd_hbm, out_hbm,
            src_v, dst_v, buf_v, agg_sh, sem):
    c = lax.axis_index("c")
    s = lax.axis_index("s")
    node_base = s * NODE_ROWS_PER_TILE
    pltpu.sync_copy(zerosd_hbm.at[pl.ds(node_base, NODE_ROWS_PER_TILE)],
                    agg_sh.at[pl.ds(node_base, NODE_ROWS_PER_TILE)])
    edge_base = (c * NTILE + s) * ROWS_PER_TILE
    pltpu.sync_copy(srcr_hbm.at[pl.ds(edge_base, ROWS_PER_TILE)], src_v)
    pltpu.sync_copy(dstr_hbm.at[pl.ds(edge_base, ROWS_PER_TILE)], dst_v)
    plsc.subcore_barrier()

    def body(j, carry):
        pltpu.async_copy(y_hbm.at[src_v.at[j]], buf_v, sem).wait()
        pltpu.sync_copy(buf_v, agg_sh.at[dst_v.at[j]], add=True)
        return carry

    lax.fori_loop(0, ROWS_PER_TILE, body, 0)
    plsc.subcore_barrier()
    pltpu.sync_copy(agg_sh.at[pl.ds(node_base, NODE_ROWS_PER_TILE)],
                    out_hbm.at[c, pl.ds(node_base, NODE_ROWS_PER_TILE)])


_BM = 512
_GRID = NP // _BM


def _dis_block(h_ref):
    deg = 1.0 + h_ref[0, :, 0:1] + h_ref[1, :, 0:1]
    return lax.rsqrt(deg)


def _t1_body(x_ref, w_ref, h_ref, y_ref):
    xw = jnp.dot(x_ref[...], w_ref[...], preferred_element_type=jnp.float32)
    y_ref[...] = xw * _dis_block(h_ref)


def _t2_body(p_ref, y_ref, b_ref, w_ref, h_ref, o_ref):
    dis = _dis_block(h_ref)
    conv = (p_ref[0] + p_ref[1] + y_ref[...]) * dis + b_ref[...]
    h = jnp.maximum(conv, 0.0)
    o_ref[...] = jnp.dot(h, w_ref[...], preferred_element_type=jnp.float32) * dis


def _t3_body(q_ref, y_ref, b_ref, wfc_ref, bfc_ref, h_ref, o_ref):
    dis = _dis_block(h_ref)
    conv = (q_ref[0] + q_ref[1] + y_ref[...]) * dis + b_ref[...]
    logits = jnp.sum(conv * wfc_ref[...], axis=1, keepdims=True) + bfc_ref[0, 0]
    o_ref[...] = jax.nn.sigmoid(logits)


_spec_rows = pl.BlockSpec((_BM, D), lambda i: (i, 0))
_spec_hist = pl.BlockSpec((NSC, _BM, 16), lambda i: (0, i, 0))
_spec_part = pl.BlockSpec((NSC, _BM, D), lambda i: (0, i, 0))
_spec_w = pl.BlockSpec((D, D), lambda i: (0, 0))
_spec_b = pl.BlockSpec((1, D), lambda i: (0, 0))
_spec_s = pl.BlockSpec((1, 1), lambda i: (0, 0))

_t1 = pl.pallas_call(
    _t1_body,
    grid=(_GRID,),
    in_specs=[_spec_rows, _spec_w, _spec_hist],
    out_specs=_spec_rows,
    out_shape=jax.ShapeDtypeStruct((NP, D), jnp.float32),
)

_t2 = pl.pallas_call(
    _t2_body,
    grid=(_GRID,),
    in_specs=[_spec_part, _spec_rows, _spec_b, _spec_w, _spec_hist],
    out_specs=_spec_rows,
    out_shape=jax.ShapeDtypeStruct((NP, D), jnp.float32),
)

_t3 = pl.pallas_call(
    _t3_body,
    grid=(_GRID,),
    in_specs=[_spec_part, _spec_rows, _spec_b, _spec_b, _spec_s, _spec_hist],
    out_specs=pl.BlockSpec((_BM, 1), lambda i: (i, 0)),
    out_shape=jax.ShapeDtypeStruct((NP, 1), jnp.float32),
)


def kernel(x, edge_index, W1, b1, W2, b2, Wfc, bfc):
    pad = EP - E
    src = jnp.concatenate([edge_index[0], jnp.full((pad,), N, jnp.int32)])
    dst = jnp.concatenate([edge_index[1], jnp.full((pad,), N, jnp.int32)])
    srcr = src.reshape(NROWS, CHUNK)
    dstr = dst.reshape(NROWS, CHUNK)
    x_pad = jnp.zeros((NP, D), jnp.float32).at[:N].set(x)
    zeros16 = jnp.zeros((NP, 16), jnp.float32)
    ones16 = jnp.ones((CHUNK, 16), jnp.float32)
    zerosd = jnp.zeros((NP, D), jnp.float32)

    hist = _hist_sc(dstr, zeros16, ones16)
    y1 = _t1(x_pad, W1, hist)
    p1 = _agg_sc(y1, srcr, dstr, zerosd)
    y2 = _t2(p1, y1, b1.reshape(1, D), W2, hist)
    p2 = _agg_sc(y2, srcr, dstr, zerosd)
    out = _t3(p2, y2, b2.reshape(1, D), Wfc.reshape(1, D), bfc.reshape(1, 1),
              hist)
    return out[:N]


# speedup vs baseline: 8.2876x; 8.2876x over previous
import functools

import jax
import jax.numpy as jnp
from jax import lax
from jax.experimental import pallas as pl
from jax.experimental.pallas import tpu as pltpu
from jax.experimental.pallas import tpu_sc as plsc

N = 10000
E = 320000
D = 128

NP = 10240
CHUNK = 128
EP = 327680
NROWS = EP // CHUNK
NSC = 2
NTILE = 16
ROWS_PER_TILE = NROWS // (NSC * NTILE)
NODE_ROWS_PER_TILE = NP // NTILE

def _hist_body(dstr_hbm, zeros16_hbm, ones16_hbm, out_hbm, dst_v, ones_v, deg_sh):
    c = lax.axis_index("c")
    s = lax.axis_index("s")
    node_base = s * NODE_ROWS_PER_TILE
    pltpu.sync_copy(zeros16_hbm.at[pl.ds(node_base, NODE_ROWS_PER_TILE)],
                    deg_sh.at[pl.ds(node_base, NODE_ROWS_PER_TILE)])
    pltpu.sync_copy(ones16_hbm, ones_v)
    edge_base = (c * NTILE + s) * ROWS_PER_TILE
    pltpu.sync_copy(dstr_hbm.at[pl.ds(edge_base, ROWS_PER_TILE)], dst_v)
    plsc.subcore_barrier()

    def body(j, carry):
        pltpu.sync_copy(ones_v, deg_sh.at[dst_v.at[j]], add=True)
        return carry

    lax.fori_loop(0, ROWS_PER_TILE, body, 0)
    plsc.subcore_barrier()
    pltpu.sync_copy(deg_sh.at[pl.ds(node_base, NODE_ROWS_PER_TILE)],
                    out_hbm.at[c, pl.ds(node_base, NODE_ROWS_PER_TILE)])


def _agg_body(y_hbm, srcr_hbm, dstr_hbm, zerosd_hbm, out_hbm,
              src_v, dst_v, buf_v, agg_sh, sem):
    c = lax.axis_index("c")
    s = lax.axis_index("s")
    node_base = s * NODE_ROWS_PER_TILE
    pltpu.sync_copy(zerosd_hbm.at[pl.ds(node_base, NODE_ROWS_PER_TILE)],
                    agg_sh.at[pl.ds(node_base, NODE_ROWS_PER_TILE)])
    edge_base = (c * NTILE + s) * ROWS_PER_TILE
    pltpu.sync_copy(srcr_hbm.at[pl.ds(edge_base, ROWS_PER_TILE)], src_v)
    pltpu.sync_copy(dstr_hbm.at[pl.ds(edge_base, ROWS_PER_TILE)], dst_v)
    plsc.subcore_barrier()

    def body(j, carry):
        pltpu.async_copy(y_hbm.at[src_v.at[j]], buf_v, sem).wait()
        pltpu.sync_copy(buf_v, agg_sh.at[dst_v.at[j]], add=True)
        return carry

    lax.fori_loop(0, ROWS_PER_TILE, body, 0)
    plsc.subcore_barrier()
    pltpu.sync_copy(agg_sh.at[pl.ds(node_base, NODE_ROWS_PER_TILE)],
                    out_hbm.at[c, pl.ds(node_base, NODE_ROWS_PER_TILE)])


@functools.cache
def _sc_kernels():
    mesh = plsc.VectorSubcoreMesh(core_axis_name="c", subcore_axis_name="s")
    hist_sc = pl.kernel(
        _hist_body,
        mesh=mesh,
        out_type=jax.ShapeDtypeStruct((NSC, NP, 16), jnp.float32),
        scratch_types=[
            pltpu.VMEM((ROWS_PER_TILE, CHUNK), jnp.int32),
            pltpu.VMEM((CHUNK, 16), jnp.float32),
            pltpu.VMEM_SHARED((NP, 16), jnp.float32),
        ],
    )
    agg_sc = pl.kernel(
        _agg_body,
        mesh=mesh,
        out_type=jax.ShapeDtypeStruct((NSC, NP, D), jnp.float32),
        scratch_types=[
            pltpu.VMEM((ROWS_PER_TILE, CHUNK), jnp.int32),
            pltpu.VMEM((ROWS_PER_TILE, CHUNK), jnp.int32),
            pltpu.VMEM((CHUNK, D), jnp.float32),
            pltpu.VMEM_SHARED((NP, D), jnp.float32),
            pltpu.SemaphoreType.DMA,
        ],
    )
    return hist_sc, agg_sc


_BM = 512
_GRID = NP // _BM


def _dis_block(h_ref):
    deg = 1.0 + h_ref[0, :, 0:1] + h_ref[1, :, 0:1]
    return lax.rsqrt(deg)


def _t1_body(x_ref, w_ref, h_ref, y_ref):
    xw = jnp.dot(x_ref[...], w_ref[...], preferred_element_type=jnp.float32)
    y_ref[...] = xw * _dis_block(h_ref)


def _t2_body(p_ref, y_ref, b_ref, w_ref, h_ref, o_ref):
    dis = _dis_block(h_ref)
    conv = (p_ref[0] + p_ref[1] + y_ref[...]) * dis + b_ref[...]
    h = jnp.maximum(conv, 0.0)
    o_ref[...] = jnp.dot(h, w_ref[...], preferred_element_type=jnp.float32) * dis


def _t3_body(q_ref, y_ref, b_ref, wfc_ref, bfc_ref, h_ref, o_ref):
    dis = _dis_block(h_ref)
    conv = (q_ref[0] + q_ref[1] + y_ref[...]) * dis + b_ref[...]
    logits = jnp.sum(conv * wfc_ref[...], axis=1, keepdims=True) + bfc_ref[0, 0]
    o_ref[...] = jax.nn.sigmoid(logits)


_spec_rows = pl.BlockSpec((_BM, D), lambda i: (i, 0))
_spec_hist = pl.BlockSpec((NSC, _BM, 16), lambda i: (0, i, 0))
_spec_part = pl.BlockSpec((NSC, _BM, D), lambda i: (0, i, 0))
_spec_w = pl.BlockSpec((D, D), lambda i: (0, 0))
_spec_b = pl.BlockSpec((1, D), lambda i: (0, 0))
_spec_s = pl.BlockSpec((1, 1), lambda i: (0, 0))

_t1 = pl.pallas_call(
    _t1_body,
    grid=(_GRID,),
    in_specs=[_spec_rows, _spec_w, _spec_hist],
    out_specs=_spec_rows,
    out_shape=jax.ShapeDtypeStruct((NP, D), jnp.float32),
)

_t2 = pl.pallas_call(
    _t2_body,
    grid=(_GRID,),
    in_specs=[_spec_part, _spec_rows, _spec_b, _spec_w, _spec_hist],
    out_specs=_spec_rows,
    out_shape=jax.ShapeDtypeStruct((NP, D), jnp.float32),
)

_t3 = pl.pallas_call(
    _t3_body,
    grid=(_GRID,),
    in_specs=[_spec_part, _spec_rows, _spec_b, _spec_b, _spec_s, _spec_hist],
    out_specs=pl.BlockSpec((_BM, 1), lambda i: (i, 0)),
    out_shape=jax.ShapeDtypeStruct((NP, 1), jnp.float32),
)


def kernel(x, edge_index, W1, b1, W2, b2, Wfc, bfc):
    pad = EP - E
    src = jnp.concatenate([edge_index[0], jnp.full((pad,), N, jnp.int32)])
    dst = jnp.concatenate([edge_index[1], jnp.full((pad,), N, jnp.int32)])
    srcr = src.reshape(NROWS, CHUNK)
    dstr = dst.reshape(NROWS, CHUNK)
    x_pad = jnp.zeros((NP, D), jnp.float32).at[:N].set(x)
    zeros16 = jnp.zeros((NP, 16), jnp.float32)
    ones16 = jnp.ones((CHUNK, 16), jnp.float32)
    zerosd = jnp.zeros((NP, D), jnp.float32)

    hist_sc, agg_sc = _sc_kernels()
    hist = hist_sc(dstr, zeros16, ones16)
    y1 = _t1(x_pad, W1, hist)
    p1 = agg_sc(y1, srcr, dstr, zerosd)
    y2 = _t2(p1, y1, b1.reshape(1, D), W2, hist)
    p2 = agg_sc(y2, srcr, dstr, zerosd)
    out = _t3(p2, y2, b2.reshape(1, D), Wfc.reshape(1, D), bfc.reshape(1, 1),
              hist)
    return out[:N]


# speedup vs baseline: 8.6847x; 1.0479x over previous
import functools

import jax
import jax.numpy as jnp
from jax import lax
from jax.experimental import pallas as pl
from jax.experimental.pallas import tpu as pltpu
from jax.experimental.pallas import tpu_sc as plsc

N = 10000
E = 320000
D = 128

NP = 10240
EP = 327680
NSC = 2
NTILE = 16

HCHUNK = 128
HROWS = EP // HCHUNK
HIST_ROWS_PER_TILE = HROWS // (NSC * NTILE)
HIST_NODE_ROWS = NP // NTILE

CHUNK = 128
NROWS = EP // CHUNK
AGG_ROWS_PER_TILE = NROWS // (NSC * NTILE)
TBL = 10112
NODE_ROWS_PER_TILE = TBL // NTILE

NBUF = 2
LEAD = 1
IRING = 8


def _hist_body(dstr_hbm, zeros16_hbm, ones16_hbm, out_hbm, dst_v, ones_v, deg_sh):
    c = lax.axis_index("c")
    s = lax.axis_index("s")
    node_base = s * HIST_NODE_ROWS
    pltpu.sync_copy(zeros16_hbm.at[pl.ds(node_base, HIST_NODE_ROWS)],
                    deg_sh.at[pl.ds(node_base, HIST_NODE_ROWS)])
    pltpu.sync_copy(ones16_hbm, ones_v)
    edge_base = (c * NTILE + s) * HIST_ROWS_PER_TILE
    pltpu.sync_copy(dstr_hbm.at[pl.ds(edge_base, HIST_ROWS_PER_TILE)], dst_v)
    plsc.subcore_barrier()

    def body(j, carry):
        pltpu.sync_copy(ones_v, deg_sh.at[dst_v.at[j]], add=True)
        return carry

    lax.fori_loop(0, HIST_ROWS_PER_TILE, body, 0)
    plsc.subcore_barrier()
    pltpu.sync_copy(deg_sh.at[pl.ds(node_base, HIST_NODE_ROWS)],
                    out_hbm.at[c, pl.ds(node_base, HIST_NODE_ROWS)])


def _agg_body(y_hbm, srcr_hbm, dstr_hbm, zerosd_hbm, out_hbm,
              src_v, dst_v, buf_v, agg_sh, *sems):
    gsem = sems[:NBUF]
    isem = sems[NBUF:]
    c = lax.axis_index("c")
    s = lax.axis_index("s")
    node_base = s * NODE_ROWS_PER_TILE
    pltpu.sync_copy(zerosd_hbm.at[pl.ds(node_base, NODE_ROWS_PER_TILE)],
                    agg_sh.at[pl.ds(node_base, NODE_ROWS_PER_TILE)])
    edge_base = (c * NTILE + s) * AGG_ROWS_PER_TILE

    def prefetch_idx(j, slot):
        sl = pl.ds((edge_base + j) * CHUNK, CHUNK)
        pltpu.async_copy(srcr_hbm.at[sl], src_v.at[slot], isem[slot])
        pltpu.async_copy(dstr_hbm.at[sl], dst_v.at[slot], isem[slot])

    def idx_wait(j, slot):
        sl = pl.ds((edge_base + j) * CHUNK, CHUNK)
        pltpu.make_async_copy(srcr_hbm.at[sl], src_v.at[slot],
                              isem[slot]).wait()
        pltpu.make_async_copy(dstr_hbm.at[sl], dst_v.at[slot],
                              isem[slot]).wait()

    def gather(j, slot):
        pltpu.async_copy(y_hbm.at[src_v.at[slot]], buf_v.at[slot], gsem[slot])

    def gather_wait(j, slot):
        pltpu.make_async_copy(y_hbm.at[src_v.at[slot]], buf_v.at[slot],
                              gsem[slot]).wait()

    def scatter(j, slot):
        pltpu.sync_copy(buf_v.at[slot], agg_sh.at[dst_v.at[slot]], add=True)

    for m in range(NBUF):
        prefetch_idx(m, m)
    for b in range(LEAD):
        idx_wait(b, b)
        gather(b, b)

    def step(j, b, when):
        gather_wait(j, b)
        scatter(j, b)
        pre = j + NBUF
        when(pre < AGG_ROWS_PER_TILE, lambda: prefetch_idx(pre, b))
        nxt = j + LEAD
        ns = (b + LEAD) % NBUF

        def _issue():
            idx_wait(nxt, ns)
            gather(nxt, ns)

        when(nxt < AGG_ROWS_PER_TILE, _issue)

    def when_static(cond, f):
        if cond:
            f()

    def when_traced(cond, f):
        pl.when(cond)(f)

    for b in range(NBUF):
        step(b, b, when_static)

    def group(g, carry):
        for b in range(NBUF):
            step(g * NBUF + b, b, when_traced)
        return carry

    lax.fori_loop(1, AGG_ROWS_PER_TILE // NBUF, group, 0)
    plsc.subcore_barrier()
    pltpu.sync_copy(agg_sh.at[pl.ds(node_base, NODE_ROWS_PER_TILE)],
                    out_hbm.at[c, pl.ds(node_base, NODE_ROWS_PER_TILE)])


@functools.cache
def _sc_kernels():
    mesh = plsc.VectorSubcoreMesh(core_axis_name="c", subcore_axis_name="s")
    hist_sc = pl.kernel(
        _hist_body,
        mesh=mesh,
        out_type=jax.ShapeDtypeStruct((NSC, NP, 16), jnp.float32),
        scratch_types=[
            pltpu.VMEM((HIST_ROWS_PER_TILE, HCHUNK), jnp.int32),
            pltpu.VMEM((HCHUNK, 16), jnp.float32),
            pltpu.VMEM_SHARED((NP, 16), jnp.float32),
        ],
    )
    agg_sc = pl.kernel(
        _agg_body,
        mesh=mesh,
        out_type=jax.ShapeDtypeStruct((NSC, NP, D), jnp.float32),
        scratch_types=[
            pltpu.VMEM((IRING, CHUNK), jnp.int32),
            pltpu.VMEM((IRING, CHUNK), jnp.int32),
            pltpu.VMEM((NBUF, CHUNK, D), jnp.float32),
            pltpu.VMEM_SHARED((TBL, D), jnp.float32),
        ] + [pltpu.SemaphoreType.DMA] * (2 * NBUF),
    )
    return hist_sc, agg_sc


_BM = 512
_GRID = NP // _BM


def _dis_block(h_ref):
    deg = 1.0 + h_ref[0, :, 0:1] + h_ref[1, :, 0:1]
    return lax.rsqrt(deg)


def _t1_body(x_ref, w_ref, h_ref, y_ref):
    xw = jnp.dot(x_ref[...], w_ref[...], preferred_element_type=jnp.float32)
    y_ref[...] = xw * _dis_block(h_ref)


def _t2_body(p_ref, y_ref, b_ref, w_ref, h_ref, o_ref):
    dis = _dis_block(h_ref)
    conv = (p_ref[0] + p_ref[1] + y_ref[...]) * dis + b_ref[...]
    h = jnp.maximum(conv, 0.0)
    o_ref[...] = jnp.dot(h, w_ref[...], preferred_element_type=jnp.float32) * dis


def _t3_body(q_ref, y_ref, b_ref, wfc_ref, bfc_ref, h_ref, o_ref):
    dis = _dis_block(h_ref)
    conv = (q_ref[0] + q_ref[1] + y_ref[...]) * dis + b_ref[...]
    logits = jnp.sum(conv * wfc_ref[...], axis=1, keepdims=True) + bfc_ref[0, 0]
    o_ref[...] = jax.nn.sigmoid(logits)


_spec_rows = pl.BlockSpec((_BM, D), lambda i: (i, 0))
_spec_part = pl.BlockSpec((NSC, _BM, D), lambda i: (0, i, 0))
_spec_hist = pl.BlockSpec((NSC, _BM, 16), lambda i: (0, i, 0))
_spec_w = pl.BlockSpec((D, D), lambda i: (0, 0))
_spec_b = pl.BlockSpec((1, D), lambda i: (0, 0))
_spec_s = pl.BlockSpec((1, 1), lambda i: (0, 0))

_t1 = pl.pallas_call(
    _t1_body,
    grid=(_GRID,),
    in_specs=[_spec_rows, _spec_w, _spec_hist],
    out_specs=_spec_rows,
    out_shape=jax.ShapeDtypeStruct((NP, D), jnp.float32),
)

_t2 = pl.pallas_call(
    _t2_body,
    grid=(_GRID,),
    in_specs=[_spec_part, _spec_rows, _spec_b, _spec_w, _spec_hist],
    out_specs=_spec_rows,
    out_shape=jax.ShapeDtypeStruct((NP, D), jnp.float32),
)

_t3 = pl.pallas_call(
    _t3_body,
    grid=(_GRID,),
    in_specs=[_spec_part, _spec_rows, _spec_b, _spec_b, _spec_s, _spec_hist],
    out_specs=pl.BlockSpec((_BM, 1), lambda i: (i, 0)),
    out_shape=jax.ShapeDtypeStruct((NP, 1), jnp.float32),
)


def kernel(x, edge_index, W1, b1, W2, b2, Wfc, bfc):
    pad = EP - E
    src = jnp.concatenate([edge_index[0], jnp.full((pad,), N, jnp.int32)])
    dst = jnp.concatenate([edge_index[1], jnp.full((pad,), N, jnp.int32)])
    srcr = src
    dstr = dst
    dstr_h = dst.reshape(HROWS, HCHUNK)
    x_pad = jnp.zeros((NP, D), jnp.float32).at[:N].set(x)
    zeros16 = jnp.zeros((NP, 16), jnp.float32)
    ones16 = jnp.ones((HCHUNK, 16), jnp.float32)
    zerosd = jnp.zeros((NP, D), jnp.float32)

    hist_sc, agg_sc = _sc_kernels()
    hist = hist_sc(dstr_h, zeros16, ones16)
    y1 = _t1(x_pad, W1, hist)
    p1 = agg_sc(y1, srcr, dstr, zerosd)
    y2 = _t2(p1, y1, b1.reshape(1, D), W2, hist)
    p2 = agg_sc(y2, srcr, dstr, zerosd)
    out = _t3(p2, y2, b2.reshape(1, D), Wfc.reshape(1, D), bfc.reshape(1, 1),
              hist)
    return out[:N]
